# baseline (speedup 1.0000x reference)
import functools

import jax
import jax.numpy as jnp
from jax import lax
from jax.experimental import pallas as pl
from jax.experimental.pallas import tpu as pltpu
from jax.experimental.pallas import tpu_sc as plsc

D = 16
L = 50
G = 8
W = G * D


def _sc_gather(u2e, v2e, r2e_pad, hv, hu, sa, vr, ur, nu, nv):
    BL = hv.shape[0]
    B = nu.shape[0]
    info = plsc.get_sparse_core_info()
    NC, NS = info.num_cores, info.num_subcores
    NW = NC * NS
    big = BL // NW
    small = B // NW
    mesh = plsc.VectorSubcoreMesh(core_axis_name="c", subcore_axis_name="s")

    @functools.partial(
        pl.kernel,
        out_type=[jax.ShapeDtypeStruct((BL, D), jnp.float32)] * 5
        + [jax.ShapeDtypeStruct((B, D), jnp.float32)] * 2,
        mesh=mesh,
        scratch_types=[
            pltpu.VMEM((big,), jnp.int32),
            pltpu.VMEM((big, D), jnp.float32),
            pltpu.VMEM((small,), jnp.int32),
            pltpu.VMEM((small, D), jnp.float32),
            pltpu.SemaphoreType.DMA,
        ],
        compiler_params=pltpu.CompilerParams(use_tc_tiling_on_sc=False),
    )
    def k(u2e_h, v2e_h, r2e_h, hv_h, hu_h, sa_h, vr_h, ur_h, nu_h, nv_h,
          o_pt, o_qa, o_un, o_eri, o_eru, o_pi, o_qj,
          idx_v, rows_v, idx_s, rows_s, sem):
        wid = lax.axis_index("s") * NC + lax.axis_index("c")
        base = wid * big
        for tab, ih, oh in ((u2e_h, hv_h, o_pt), (v2e_h, hu_h, o_qa),
                            (u2e_h, sa_h, o_un), (r2e_h, vr_h, o_eri),
                            (r2e_h, ur_h, o_eru)):
            pltpu.sync_copy(ih.at[pl.ds(base, big)], idx_v)
            pltpu.async_copy(tab.at[idx_v], rows_v, sem).wait()
            pltpu.sync_copy(rows_v, oh.at[pl.ds(base, big)])
        sbase = wid * small
        for tab, ih, oh in ((u2e_h, nu_h, o_pi), (v2e_h, nv_h, o_qj)):
            pltpu.sync_copy(ih.at[pl.ds(sbase, small)], idx_s)
            pltpu.async_copy(tab.at[idx_s], rows_s, sem).wait()
            pltpu.sync_copy(rows_s, oh.at[pl.ds(sbase, small)])

    return k(u2e, v2e, r2e_pad, hv, hu, sa, vr, ur, nu, nv)


def _relu(x):
    return jnp.maximum(x, 0.0)


def _mm(x, w):
    return jax.lax.dot(x, w, preferred_element_type=jnp.float32)


def _mme(x, w):
    return jax.lax.dot(x, w, precision=jax.lax.Precision.HIGHEST,
                       preferred_element_type=jnp.float32)


def _tc_main(pt, qa, un, eri, eru, qj, pi, rep16, wts, B, BB):
    grid = (B // G) // BB
    n2 = L * BB

    def body(pt_r, qa_r, un_r, eri_r, eru_r, qj_r, pi_r, rep16_r,
             gi_w1a, gi_w1b, gi_b1, gi_w2, gi_b2, gi_w3, gi_b3,
             ai_w1a, ai_w1b, ai_b1, ai_w2, ai_b2, ai_w3, ai_b3,
             gu_w1a, gu_w1b, gu_b1, gu_w2, gu_b2, gu_w3, gu_b3,
             au_w1a, au_w1b, au_b1, au_w2, au_b2, au_w3, au_b3,
             as_w1a, as_w1b, as_b1, as_w2, as_b2, as_w3, as_b3,
             mu_w1a, mu_w1b, mu_b1, mu_w2, mu_b2, mu_w3, mu_b3,
             ir1_w, ir1_b, ir2_w, ir2_b,
             hi_o, zj_o):
        rep16 = rep16_r[...]

        def gv_mlp(x, er, w1a, w1b, b1, w2, b2, w3, b3):
            h = _relu(_mm(x, w1a[...]) + _mm(er, w1b[...]) + b1[...])
            h = _relu(_mm(h, w2[...]) + b2[...])
            return _mm(h, w3[...]) + b3[...]

        def attn_agg(neigh, rep, w1a, w1b, b1, w2, b2, w3, b3):
            repp = _mm(rep, w1b[...]) + b1[...]
            a3 = _mm(neigh, w1a[...]).reshape(L, BB, W) + repp[None]
            h = _relu(a3).reshape(n2, W)
            h = _relu(_mm(h, w2[...]) + b2[...])
            s = (_mm(h, w3[...]) + b3[...]).reshape(L, BB, G)
            mu = jax.nn.softmax(s, axis=0)
            mu_exp = _mme(mu.reshape(n2, G), rep16)
            return jnp.sum((neigh * mu_exp).reshape(L, BB, W), axis=0)

        qjv = qj_r[...]
        piv = pi_r[...]

        ptv = pt_r[...].reshape(n2, W)
        fjt = gv_mlp(ptv, eri_r[...].reshape(n2, W), gi_w1a, gi_w1b,
                     gi_b1, gi_w2, gi_b2, gi_w3, gi_b3)
        zj = attn_agg(fjt, qjv, ai_w1a, ai_w1b, ai_b1, ai_w2, ai_b2,
                      ai_w3, ai_b3)
        zj = _relu(_mm(zj, ir1_w[...]) + ir1_b[...])
        zj = _relu(_mm(zj, ir2_w[...]) + ir2_b[...])
        zj_o[...] = zj

        qav = qa_r[...].reshape(n2, W)
        xia = gv_mlp(qav, eru_r[...].reshape(n2, W), gu_w1a, gu_w1b,
                     gu_b1, gu_w2, gu_b2, gu_w3, gu_b3)
        hi_i = attn_agg(xia, piv, au_w1a, au_w1b, au_b1, au_w2, au_b2,
                        au_w3, au_b3)

        unv = un_r[...].reshape(n2, W)
        hi_s = attn_agg(unv, piv, as_w1a, as_w1b, as_b1, as_w2, as_b2,
                        as_w3, as_b3)

        h = _relu(_mm(hi_i, mu_w1a[...]) + _mm(hi_s, mu_w1b[...])
                  + mu_b1[...])
        h = _relu(_mm(h, mu_w2[...]) + mu_b2[...])
        hi_o[...] = _mm(h, mu_w3[...]) + mu_b3[...]

    big_spec = pl.BlockSpec((L, BB, W), lambda i: (0, i, 0))
    row_spec = pl.BlockSpec((BB, W), lambda i: (i, 0))

    def full_spec(a):
        return pl.BlockSpec(a.shape, lambda i, _r=a.ndim: (0,) * _r)

    in_specs = ([big_spec] * 5 + [row_spec, row_spec, full_spec(rep16)]
                + [full_spec(w) for w in wts])
    out_specs = [pl.BlockSpec((BB, W), lambda i: (i, 0))] * 2

    hi, zj = pl.pallas_call(
        body,
        grid=(grid,),
        in_specs=in_specs,
        out_specs=out_specs,
        out_shape=[jax.ShapeDtypeStruct((B // G, W), jnp.float32)] * 2,
    )(pt, qa, un, eri, eru, qj, pi, rep16, *wts)
    return hi, zj


def _tc_head(hi, zj, gcomb, wts, B):
    def body(hi_r, zj_r, gcomb_r,
             ur1_w, ur1_b, ur2_w, ur2_b, vr1_w, vr1_b, vr2_w, vr2_b,
             uv1_wa, uv1_wb, uv1_b, uv2_w, uv2_b, uv3_w, uv3_b,
             bn1_g, bn1_b, bn2_g, bn2_b, bn3_g, bn3_b, bn4_g, bn4_b,
             out_r):
        gcomb = gcomb_r[...]

        def bn(x, g, b, eps=1e-5):
            m = _mme(jnp.mean(x, axis=0, keepdims=True), gcomb)
            xc = x - m
            v = _mme(jnp.mean(xc * xc, axis=0, keepdims=True), gcomb)
            return g[...] * xc / jnp.sqrt(v + eps) + b[...]

        x_u = _relu(bn(_mm(hi_r[...], ur1_w[...]) + ur1_b[...], bn1_g, bn1_b))
        x_u = _mm(x_u, ur2_w[...]) + ur2_b[...]
        x_v = _relu(bn(_mm(zj_r[...], vr1_w[...]) + vr1_b[...], bn2_g, bn2_b))
        x_v = _mm(x_v, vr2_w[...]) + vr2_b[...]
        x = _relu(bn(_mm(x_u, uv1_wa[...]) + _mm(x_v, uv1_wb[...])
                     + uv1_b[...], bn3_g, bn3_b))
        x = _relu(bn(_mm(x, uv2_w[...]) + uv2_b[...], bn4_g, bn4_b))
        out_r[...] = _mm(x, uv3_w[...]) + uv3_b[...]

    out = pl.pallas_call(
        body,
        out_shape=jax.ShapeDtypeStruct((B // G, G), jnp.float32),
    )(hi, zj, gcomb, *wts)
    return out


def kernel(nodes_u, nodes_v, history_u, history_ur, social_adj, history_v,
           history_vr, u2e, v2e, r2e, gv_item, att_item, w_ir1, w_ir2,
           gv_user, att_user, att_soc, mlp_user, w_ur1, w_ur2, w_vr1, w_vr2,
           w_uv1, w_uv2, w_uv3, bn1, bn2, bn3, bn4):
    B = nodes_u.shape[0]
    BB = 64

    i32 = lambda a: jnp.asarray(a, jnp.int32)
    REP = 8192
    r2e_pad = jnp.zeros((G, D), jnp.float32).at[:r2e.shape[0]].set(r2e)
    r2e_rep = jnp.tile(r2e_pad, (REP, 1))

    C = 2
    Bc = B // C
    spread = (jnp.arange(Bc * L, dtype=jnp.int32) % REP) * G

    def chunk_gather(c):
        sl = slice(c * Bc, (c + 1) * Bc)
        t = lambda a: i32(a[sl]).T.reshape(-1)
        return _sc_gather(u2e, v2e, r2e_rep, t(history_v), t(history_u),
                          t(social_adj), t(history_vr) + spread,
                          t(history_ur) + spread, i32(nodes_u[sl]),
                          i32(nodes_v[sl]))

    gathered = [chunk_gather(c) for c in range(C)]
    pack3 = lambda a: a.reshape(L, Bc // G, W)
    packr = lambda a: a.reshape(Bc // G, W)

    eye8 = jnp.eye(G, dtype=jnp.float32)
    kron8 = lambda w: jnp.kron(eye8, w)
    tile8 = lambda b: jnp.tile(b.reshape(1, -1), (1, G))

    def split_mlp(p):
        w1, b1, w2, b2, w3, b3 = p
        return [kron8(w1[:D]), kron8(w1[D:]), tile8(b1),
                kron8(w2), tile8(b2), kron8(w3), tile8(b3)]

    def split_att(p):
        w1, b1, w2, b2, w3, b3 = p
        return [kron8(w1[:D]), kron8(w1[D:]), tile8(b1),
                kron8(w2), tile8(b2), kron8(w3),
                jnp.full((1, G), b3[0], jnp.float32)]

    rep16 = jnp.kron(eye8, jnp.ones((1, D), jnp.float32))
    gcomb = jnp.kron(jnp.full((G, G), 1.0 / G, jnp.float32),
                     jnp.eye(D, dtype=jnp.float32))

    main_wts = (split_mlp(gv_item) + split_att(att_item)
                + split_mlp(gv_user) + split_att(att_user)
                + split_att(att_soc) + split_mlp(mlp_user)
                + [kron8(w_ir1[0]), tile8(w_ir1[1]),
                   kron8(w_ir2[0]), tile8(w_ir2[1])])

    his, zjs = [], []
    for pt, qa, un, eri, eru, pi, qj in gathered:
        hi_c, zj_c = _tc_main(pack3(pt), pack3(qa), pack3(un), pack3(eri),
                              pack3(eru), packr(qj), packr(pi), rep16,
                              main_wts, Bc, BB)
        his.append(hi_c)
        zjs.append(zj_c)
    hi = jnp.concatenate(his, axis=0)
    zj = jnp.concatenate(zjs, axis=0)

    head_wts = [kron8(w_ur1[0]), tile8(w_ur1[1]),
                kron8(w_ur2[0]), tile8(w_ur2[1]),
                kron8(w_vr1[0]), tile8(w_vr1[1]),
                kron8(w_vr2[0]), tile8(w_vr2[1]),
                kron8(w_uv1[0][:D]), kron8(w_uv1[0][D:]), tile8(w_uv1[1]),
                kron8(w_uv2[0]), tile8(w_uv2[1]),
                kron8(w_uv3[0]), jnp.full((1, G), w_uv3[1][0], jnp.float32),
                tile8(bn1[0]), tile8(bn1[1]), tile8(bn2[0]), tile8(bn2[1]),
                tile8(bn3[0]), tile8(bn3[1]), tile8(bn4[0]), tile8(bn4[1])]

    scores = _tc_head(hi, zj, gcomb, head_wts, B)
    return scores.reshape(B)


# speedup vs baseline: 1.0780x; 1.0780x over previous
import functools

import jax
import jax.numpy as jnp
from jax import lax
from jax.experimental import pallas as pl
from jax.experimental.pallas import tpu as pltpu
from jax.experimental.pallas import tpu_sc as plsc

D = 16
L = 50
G = 8
W = G * D
NW = 32


def _sc_gather(u2e, v2e, r2e_rep, hv, hu, sa, vr, ur, nu, nv):
    BL = hv.shape[0]
    B = nu.shape[0]
    big = BL // NW
    small = B // NW
    bpw = B // NW
    shift = bpw.bit_length() - 1
    mesh = plsc.VectorSubcoreMesh(core_axis_name="c", subcore_axis_name="s")

    @functools.partial(
        pl.kernel,
        out_type=[jax.ShapeDtypeStruct((BL, D), jnp.float32)] * 5
        + [jax.ShapeDtypeStruct((B, D), jnp.float32)] * 2,
        mesh=mesh,
        scratch_types=[
            pltpu.VMEM((big,), jnp.int32),
            pltpu.VMEM((big,), jnp.int32),
            pltpu.VMEM((big, D), jnp.float32),
            pltpu.VMEM((small,), jnp.int32),
            pltpu.VMEM((small, D), jnp.float32),
            pltpu.SemaphoreType.DMA,
        ],
        compiler_params=pltpu.CompilerParams(use_tc_tiling_on_sc=False,
                                             needs_layout_passes=False),
    )
    def k(u2e_h, v2e_h, r2e_h, hv_h, hu_h, sa_h, vr_h, ur_h, nu_h, nv_h,
          o_pt, o_qa, o_un, o_eri, o_eru, o_pi, o_qj,
          idx_raw, idx_v, rows_v, idx_s, rows_s, sem):
        wid = lax.axis_index("s") * 2 + lax.axis_index("c")
        base = wid * big
        lanes = lax.iota(jnp.int32, 16)

        def reorder(kk, _):
            p = kk * 16 + lanes
            db = jnp.bitwise_and(p, bpw - 1)
            l = jnp.right_shift(p, shift)
            vals = plsc.load_gather(idx_raw, [db * L + l])
            idx_v[pl.ds(kk * 16, 16)] = vals
            return 0

        for tab, ih, oh in ((u2e_h, hv_h, o_pt), (v2e_h, hu_h, o_qa),
                            (u2e_h, sa_h, o_un), (r2e_h, vr_h, o_eri),
                            (r2e_h, ur_h, o_eru)):
            pltpu.sync_copy(ih.at[pl.ds(base, big)], idx_raw)
            lax.fori_loop(0, big // 16, reorder, 0, unroll=8)
            pltpu.async_copy(tab.at[idx_v], rows_v, sem).wait()
            pltpu.sync_copy(rows_v, oh.at[pl.ds(base, big)])
        sbase = wid * small
        for tab, ih, oh in ((u2e_h, nu_h, o_pi), (v2e_h, nv_h, o_qj)):
            pltpu.sync_copy(ih.at[pl.ds(sbase, small)], idx_s)
            pltpu.async_copy(tab.at[idx_s], rows_s, sem).wait()
            pltpu.sync_copy(rows_s, oh.at[pl.ds(sbase, small)])

    return k(u2e, v2e, r2e_rep, hv, hu, sa, vr, ur, nu, nv)


def _relu(x):
    return jnp.maximum(x, 0.0)


def _mm(x, w):
    return jax.lax.dot(x, w, preferred_element_type=jnp.float32)


def _mme(x, w):
    return jax.lax.dot(x, w, precision=jax.lax.Precision.HIGHEST,
                       preferred_element_type=jnp.float32)


def _tc_main(pt, qa, un, eri, eru, qj, pi, rep16, wts, B, WBLK):
    T = (B // NW) // G
    grid = NW // WBLK
    n2 = WBLK * L * T

    def body(pt_r, qa_r, un_r, eri_r, eru_r, qj_r, pi_r, rep16_r,
             gi_w1a, gi_w1b, gi_b1, gi_w2, gi_b2, gi_w3, gi_b3,
             ai_w1a, ai_w1b, ai_b1, ai_w2, ai_b2, ai_w3, ai_b3,
             gu_w1a, gu_w1b, gu_b1, gu_w2, gu_b2, gu_w3, gu_b3,
             au_w1a, au_w1b, au_b1, au_w2, au_b2, au_w3, au_b3,
             as_w1a, as_w1b, as_b1, as_w2, as_b2, as_w3, as_b3,
             mu_w1a, mu_w1b, mu_b1, mu_w2, mu_b2, mu_w3, mu_b3,
             ir1_w, ir1_b, ir2_w, ir2_b,
             hi_o, zj_o):
        rep16 = rep16_r[...]

        def gv_mlp(x, er, w1a, w1b, b1, w2, b2, w3, b3):
            h = _relu(_mm(x, w1a[...]) + _mm(er, w1b[...]) + b1[...])
            h = _relu(_mm(h, w2[...]) + b2[...])
            return _mm(h, w3[...]) + b3[...]

        def attn_agg(neigh, repp, w1a, w2, b2, w3, b3):
            a4 = _mm(neigh, w1a[...]).reshape(WBLK, L, T, W) + repp
            h = _relu(a4).reshape(n2, W)
            h = _relu(_mm(h, w2[...]) + b2[...])
            s = (_mm(h, w3[...]) + b3[...]).reshape(WBLK, L, T, G)
            mu = jax.nn.softmax(s, axis=1)
            mu_exp = _mme(mu.reshape(n2, G), rep16)
            agg = jnp.sum((neigh * mu_exp).reshape(WBLK, L, T, W), axis=1)
            return agg.reshape(WBLK * T, W)

        def rep_of(x2, w1b, b1):
            return (_mm(x2, w1b[...]) + b1[...]).reshape(WBLK, 1, T, W)

        qjv = qj_r[...].reshape(WBLK * T, W)
        piv = pi_r[...].reshape(WBLK * T, W)

        ptv = pt_r[...].reshape(n2, W)
        fjt = gv_mlp(ptv, eri_r[...].reshape(n2, W), gi_w1a, gi_w1b,
                     gi_b1, gi_w2, gi_b2, gi_w3, gi_b3)
        zj = attn_agg(fjt, rep_of(qjv, ai_w1b, ai_b1),
                      ai_w1a, ai_w2, ai_b2, ai_w3, ai_b3)
        zj = _relu(_mm(zj, ir1_w[...]) + ir1_b[...])
        zj = _relu(_mm(zj, ir2_w[...]) + ir2_b[...])
        zj_o[...] = zj

        qav = qa_r[...].reshape(n2, W)
        xia = gv_mlp(qav, eru_r[...].reshape(n2, W), gu_w1a, gu_w1b,
                     gu_b1, gu_w2, gu_b2, gu_w3, gu_b3)
        hi_i = attn_agg(xia, rep_of(piv, au_w1b, au_b1),
                        au_w1a, au_w2, au_b2, au_w3, au_b3)

        unv = un_r[...].reshape(n2, W)
        hi_s = attn_agg(unv, rep_of(piv, as_w1b, as_b1),
                        as_w1a, as_w2, as_b2, as_w3, as_b3)

        h = _relu(_mm(hi_i, mu_w1a[...]) + _mm(hi_s, mu_w1b[...])
                  + mu_b1[...])
        h = _relu(_mm(h, mu_w2[...]) + mu_b2[...])
        hi_o[...] = _mm(h, mu_w3[...]) + mu_b3[...]

    big_spec = pl.BlockSpec((WBLK, L * T, W), lambda i: (i, 0, 0))
    row_spec = pl.BlockSpec((WBLK, T, W), lambda i: (i, 0, 0))

    def full_spec(a):
        return pl.BlockSpec(a.shape, lambda i, _r=a.ndim: (0,) * _r)

    in_specs = ([big_spec] * 5 + [row_spec, row_spec, full_spec(rep16)]
                + [full_spec(w) for w in wts])
    out_specs = [pl.BlockSpec((WBLK * T, W), lambda i: (i, 0))] * 2

    hi, zj = pl.pallas_call(
        body,
        grid=(grid,),
        in_specs=in_specs,
        out_specs=out_specs,
        out_shape=[jax.ShapeDtypeStruct((B // G, W), jnp.float32)] * 2,
    )(pt, qa, un, eri, eru, qj, pi, rep16, *wts)
    return hi, zj


def _tc_head(hi, zj, gcomb, wts, B):
    def body(hi_r, zj_r, gcomb_r,
             ur1_w, ur1_b, ur2_w, ur2_b, vr1_w, vr1_b, vr2_w, vr2_b,
             uv1_wa, uv1_wb, uv1_b, uv2_w, uv2_b, uv3_w, uv3_b,
             bn1_g, bn1_b, bn2_g, bn2_b, bn3_g, bn3_b, bn4_g, bn4_b,
             out_r):
        gcomb = gcomb_r[...]

        def bn(x, g, b, eps=1e-5):
            m = _mme(jnp.mean(x, axis=0, keepdims=True), gcomb)
            xc = x - m
            v = _mme(jnp.mean(xc * xc, axis=0, keepdims=True), gcomb)
            return g[...] * xc / jnp.sqrt(v + eps) + b[...]

        x_u = _relu(bn(_mm(hi_r[...], ur1_w[...]) + ur1_b[...], bn1_g, bn1_b))
        x_u = _mm(x_u, ur2_w[...]) + ur2_b[...]
        x_v = _relu(bn(_mm(zj_r[...], vr1_w[...]) + vr1_b[...], bn2_g, bn2_b))
        x_v = _mm(x_v, vr2_w[...]) + vr2_b[...]
        x = _relu(bn(_mm(x_u, uv1_wa[...]) + _mm(x_v, uv1_wb[...])
                     + uv1_b[...], bn3_g, bn3_b))
        x = _relu(bn(_mm(x, uv2_w[...]) + uv2_b[...], bn4_g, bn4_b))
        out_r[...] = _mm(x, uv3_w[...]) + uv3_b[...]

    out = pl.pallas_call(
        body,
        out_shape=jax.ShapeDtypeStruct((B // G, G), jnp.float32),
    )(hi, zj, gcomb, *wts)
    return out


def kernel(nodes_u, nodes_v, history_u, history_ur, social_adj, history_v,
           history_vr, u2e, v2e, r2e, gv_item, att_item, w_ir1, w_ir2,
           gv_user, att_user, att_soc, mlp_user, w_ur1, w_ur2, w_vr1, w_vr2,
           w_uv1, w_uv2, w_uv3, bn1, bn2, bn3, bn4):
    B = nodes_u.shape[0]
    WBLK = 4
    T = (B // NW) // G

    i32 = lambda a: jnp.asarray(a, jnp.int32)
    REP = 1024
    spread = (jnp.arange(B * L, dtype=jnp.int32) % REP) * G
    r2e_pad = jnp.zeros((G, D), jnp.float32).at[:r2e.shape[0]].set(r2e)
    r2e_rep = jnp.tile(r2e_pad, (REP, 1))

    hv = i32(history_v).reshape(-1)
    hu = i32(history_u).reshape(-1)
    sa = i32(social_adj).reshape(-1)
    vr = i32(history_vr).reshape(-1) + spread
    ur = i32(history_ur).reshape(-1) + spread
    nu = i32(nodes_u)
    nv = i32(nodes_v)

    pt, qa, un, eri, eru, pi, qj = _sc_gather(
        u2e, v2e, r2e_rep, hv, hu, sa, vr, ur, nu, nv)
    pack3 = lambda a: a.reshape(NW, L * T, W)
    packr = lambda a: a.reshape(NW, T, W)

    eye8 = jnp.eye(G, dtype=jnp.float32)
    kron8 = lambda w: jnp.kron(eye8, w)
    tile8 = lambda b: jnp.tile(b.reshape(1, -1), (1, G))

    def split_mlp(p):
        w1, b1, w2, b2, w3, b3 = p
        return [kron8(w1[:D]), kron8(w1[D:]), tile8(b1),
                kron8(w2), tile8(b2), kron8(w3), tile8(b3)]

    def split_att(p):
        w1, b1, w2, b2, w3, b3 = p
        return [kron8(w1[:D]), kron8(w1[D:]), tile8(b1),
                kron8(w2), tile8(b2), kron8(w3),
                jnp.full((1, G), b3[0], jnp.float32)]

    rep16 = jnp.kron(eye8, jnp.ones((1, D), jnp.float32))
    gcomb = jnp.kron(jnp.full((G, G), 1.0 / G, jnp.float32),
                     jnp.eye(D, dtype=jnp.float32))

    main_wts = (split_mlp(gv_item) + split_att(att_item)
                + split_mlp(gv_user) + split_att(att_user)
                + split_att(att_soc) + split_mlp(mlp_user)
                + [kron8(w_ir1[0]), tile8(w_ir1[1]),
                   kron8(w_ir2[0]), tile8(w_ir2[1])])

    hi, zj = _tc_main(pack3(pt), pack3(qa), pack3(un), pack3(eri),
                      pack3(eru), packr(qj), packr(pi), rep16, main_wts,
                      B, WBLK)

    head_wts = [kron8(w_ur1[0]), tile8(w_ur1[1]),
                kron8(w_ur2[0]), tile8(w_ur2[1]),
                kron8(w_vr1[0]), tile8(w_vr1[1]),
                kron8(w_vr2[0]), tile8(w_vr2[1]),
                kron8(w_uv1[0][:D]), kron8(w_uv1[0][D:]), tile8(w_uv1[1]),
                kron8(w_uv2[0]), tile8(w_uv2[1]),
                kron8(w_uv3[0]), jnp.full((1, G), w_uv3[1][0], jnp.float32),
                tile8(bn1[0]), tile8(bn1[1]), tile8(bn2[0]), tile8(bn2[1]),
                tile8(bn3[0]), tile8(bn3[1]), tile8(bn4[0]), tile8(bn4[1])]

    scores = _tc_head(hi, zj, gcomb, head_wts, B)
    return scores.reshape(B)
